# Initial kernel scaffold; baseline (speedup 1.0000x reference)
import jax
import jax.numpy as jnp
from jax.experimental import pallas as pl


def kernel(x, edge_index, batch, w_in, b_in, g_in, be_in, W0, as0, ad0, bias0, W1, as1, ad1, bias1, Wr0, br0, Wr1, br1, Wc1, bc1, Wc2, bc2, Wd, bd, gd, bed, Wo, bo):
    raise NotImplementedError("write your pallas kernel here")


# speedup vs baseline: 1.0162x; 1.0162x over previous
import functools

import jax
import jax.numpy as jnp
from jax.experimental import pallas as pl
from jax.experimental.pallas import tpu as pltpu

HEADS = 2
N_BLOCK = 2000


def _stage1_body(x_ref, w_in_ref, b_in_ref, g_ref, be_ref,
                 W0_ref, as0_ref, ad0_ref, Wr0_ref, br0_ref,
                 h_ref, hh0_ref, asrc_ref, adst_ref, res_ref):
    x = x_ref[...]
    w_in = w_in_ref[...]
    h = x * w_in + b_in_ref[...]
    m = jnp.mean(h, axis=-1, keepdims=True)
    v = jnp.mean((h - m) ** 2, axis=-1, keepdims=True)
    h = (h - m) * jax.lax.rsqrt(v + 1e-5) * g_ref[...] + be_ref[...]
    h = jnp.maximum(h, 0.0)
    h_ref[...] = h
    hh0 = jnp.dot(h, W0_ref[...], preferred_element_type=jnp.float32)
    hh0_ref[...] = hh0
    hh3 = hh0.reshape(h.shape[0], HEADS, 32)
    asrc_ref[...] = jnp.sum(hh3 * as0_ref[...][None], axis=-1)
    adst_ref[...] = jnp.sum(hh3 * ad0_ref[...][None], axis=-1)
    res_ref[...] = jnp.dot(h, Wr0_ref[...], preferred_element_type=jnp.float32) + br0_ref[...]


def _stage2_body(agg_ref, res_ref, bias0_ref,
                 W1_ref, as1_ref, ad1_ref, Wr1_ref, br1_ref,
                 hh1_ref, asrc_ref, adst_ref, res1_ref):
    x1 = jnp.maximum(agg_ref[...] + bias0_ref[...] + res_ref[...], 0.0)
    hh1 = jnp.dot(x1, W1_ref[...], preferred_element_type=jnp.float32)
    hh1_ref[...] = hh1
    hh3 = hh1.reshape(x1.shape[0], HEADS, 64)
    asrc_ref[...] = jnp.sum(hh3 * as1_ref[...][None], axis=-1)
    adst_ref[...] = jnp.sum(hh3 * ad1_ref[...][None], axis=-1)
    res1_ref[...] = jnp.dot(x1, Wr1_ref[...], preferred_element_type=jnp.float32) + br1_ref[...]


def _stage3_body(agg_ref, res_ref, bias1_ref, batch_ref,
                 Wc1_ref, bc1_ref, Wc2_ref, bc2_ref,
                 h_ref, clusters_ref):
    i = pl.program_id(0)
    h = jnp.maximum(agg_ref[...] + bias1_ref[...] + res_ref[...], 0.0)
    h_ref[...] = h
    a = jnp.dot(jnp.dot(h, Wc1_ref[...], preferred_element_type=jnp.float32) + bc1_ref[...],
                Wc2_ref[...], preferred_element_type=jnp.float32) + bc2_ref[...]
    a = a - jnp.max(a, axis=-1, keepdims=True)
    e = jnp.exp(a)
    assign = e / jnp.sum(e, axis=-1, keepdims=True)
    b = batch_ref[...].reshape(h.shape[0])
    onehot = (b[:, None] == jax.lax.broadcasted_iota(jnp.int32, (1, 8), 1)).astype(jnp.float32)
    w = onehot[:, :, None] * assign[:, None, :]
    w2 = w.reshape(h.shape[0], 64)
    part = jnp.dot(w2.T, h, preferred_element_type=jnp.float32)

    @pl.when(i == 0)
    def _():
        clusters_ref[...] = jnp.zeros_like(clusters_ref)

    clusters_ref[...] += part


def _dense_stage1(x, w_in, b_in, g_in, be_in, W0, as0, ad0, Wr0, br0):
    N = x.shape[0]
    grid = (N // N_BLOCK,)
    bs = lambda c: pl.BlockSpec((N_BLOCK, c), lambda i: (i, 0))
    ws = lambda shape: pl.BlockSpec(shape, lambda i: tuple(0 for _ in shape))
    return pl.pallas_call(
        _stage1_body,
        grid=grid,
        in_specs=[bs(1), ws((1, 64)), ws((64,)), ws((64,)), ws((64,)),
                  ws((64, 64)), ws((HEADS, 32)), ws((HEADS, 32)), ws((64, 64)), ws((64,))],
        out_specs=[bs(64), bs(64), bs(HEADS), bs(HEADS), bs(64)],
        out_shape=[jax.ShapeDtypeStruct((N, 64), jnp.float32),
                   jax.ShapeDtypeStruct((N, 64), jnp.float32),
                   jax.ShapeDtypeStruct((N, HEADS), jnp.float32),
                   jax.ShapeDtypeStruct((N, HEADS), jnp.float32),
                   jax.ShapeDtypeStruct((N, 64), jnp.float32)],
    )(x, w_in, b_in, g_in, be_in, W0, as0, ad0, Wr0, br0)


def _dense_stage2(agg, res, bias0, W1, as1, ad1, Wr1, br1):
    N = agg.shape[0]
    grid = (N // N_BLOCK,)
    bs = lambda c: pl.BlockSpec((N_BLOCK, c), lambda i: (i, 0))
    ws = lambda shape: pl.BlockSpec(shape, lambda i: tuple(0 for _ in shape))
    return pl.pallas_call(
        _stage2_body,
        grid=grid,
        in_specs=[bs(64), bs(64), ws((64,)),
                  ws((64, 128)), ws((HEADS, 64)), ws((HEADS, 64)), ws((64, 64)), ws((64,))],
        out_specs=[bs(128), bs(HEADS), bs(HEADS), bs(64)],
        out_shape=[jax.ShapeDtypeStruct((N, 128), jnp.float32),
                   jax.ShapeDtypeStruct((N, HEADS), jnp.float32),
                   jax.ShapeDtypeStruct((N, HEADS), jnp.float32),
                   jax.ShapeDtypeStruct((N, 64), jnp.float32)],
    )(agg, res, bias0, W1, as1, ad1, Wr1, br1)


def _dense_stage3(agg, res, bias1, batch, Wc1, bc1, Wc2, bc2):
    N = agg.shape[0]
    grid = (N // N_BLOCK,)
    bs = lambda c: pl.BlockSpec((N_BLOCK, c), lambda i: (i, 0))
    ws = lambda shape: pl.BlockSpec(shape, lambda i: tuple(0 for _ in shape))
    batch3 = batch.reshape(N // N_BLOCK, 1, N_BLOCK)
    h, clusters = pl.pallas_call(
        _stage3_body,
        grid=grid,
        in_specs=[bs(64), bs(64), ws((64,)),
                  pl.BlockSpec((1, 1, N_BLOCK), lambda i: (i, 0, 0)),
                  ws((64, 32)), ws((32,)), ws((32, 8)), ws((8,))],
        out_specs=[bs(64), ws((64, 64))],
        out_shape=[jax.ShapeDtypeStruct((N, 64), jnp.float32),
                   jax.ShapeDtypeStruct((64, 64), jnp.float32)],
    )(agg, res, bias1, batch3, Wc1, bc1, Wc2, bc2)
    return h, clusters


def _edge_softmax_agg(hh, asrc, adst, src, dst, N, out_ch):
    alpha = asrc[src] + adst[dst]
    alpha = jax.nn.leaky_relu(alpha, 0.2)
    amax = jax.ops.segment_max(alpha, dst, num_segments=N)
    amax = jnp.where(jnp.isfinite(amax), amax, 0.0)
    alpha = jnp.exp(alpha - amax[dst])
    denom = jax.ops.segment_sum(alpha, dst, num_segments=N)
    alpha = alpha / (denom[dst] + 1e-16)
    h3 = hh.reshape(N, HEADS, out_ch)
    msg = h3[src] * alpha[..., None]
    return jax.ops.segment_sum(msg, dst, num_segments=N)


def _final_head(clusters, Wd, bd, gd, bed, Wo, bo):
    c = clusters.reshape(8, 8, 64)
    emb = jnp.concatenate([c.max(axis=1), c.min(axis=1)], axis=1)
    e = jnp.dot(emb, Wd) + bd
    m = e.mean(-1, keepdims=True)
    v = ((e - m) ** 2).mean(-1, keepdims=True)
    e = (e - m) * jax.lax.rsqrt(v + 1e-5) * gd + bed
    e = jnp.where(e >= 0, e, 0.1 * e)
    return jnp.dot(e, Wo) + bo


def kernel(x, edge_index, batch, w_in, b_in, g_in, be_in, W0, as0, ad0, bias0,
           W1, as1, ad1, bias1, Wr0, br0, Wr1, br1, Wc1, bc1, Wc2, bc2,
           Wd, bd, gd, bed, Wo, bo):
    N = x.shape[0]
    sl = jnp.arange(N, dtype=edge_index.dtype)
    src = jnp.concatenate([edge_index[0], sl])
    dst = jnp.concatenate([edge_index[1], sl])

    h0, hh0, asrc0, adst0, res0 = _dense_stage1(
        x, w_in, b_in, g_in, be_in, W0, as0, ad0, Wr0, br0)
    agg0 = _edge_softmax_agg(hh0, asrc0, adst0, src, dst, N, 32)
    agg0 = agg0.reshape(N, 64)

    hh1, asrc1, adst1, res1 = _dense_stage2(agg0, res0, bias0, W1, as1, ad1, Wr1, br1)
    agg1 = _edge_softmax_agg(hh1, asrc1, adst1, src, dst, N, 64)
    agg1 = agg1.mean(axis=1)

    h2, clusters = _dense_stage3(agg1, res1, bias1, batch, Wc1, bc1, Wc2, bc2)
    return _final_head(clusters, Wd, bd, gd, bed, Wo, bo)


# speedup vs baseline: 28.2665x; 27.8164x over previous
import functools

import jax
import jax.numpy as jnp
from jax import lax
from jax.experimental import pallas as pl
from jax.experimental.pallas import tpu as pltpu
from jax.experimental.pallas import tpu_sc as plsc

HEADS = 2
N_BLOCK = 2000

N_NODES = 50000
NPAD = 50176
STRIPE = NPAD // 16
E_RAW = 800000 + N_NODES
EP = 851968
CH = 128
DUMMY = N_NODES


def _leaky_exp(t):
    return jnp.exp(jnp.where(t >= 0, t, 0.2 * t))


def _k1_body(asrc0, adst0, asrc1, adst1, src, dst,
             exp0, exp1, dh0, dh1,
             idx_s, idx_d, a_v, b_v, e_v, zb, acc0, acc1, sem):
    cid = lax.axis_index("c")
    sid = lax.axis_index("s")
    wid = sid * 2 + cid
    t_edges = EP // 32

    for k in range(448 // 16):
        zb[pl.ds(k * 16, 16)] = jnp.zeros((16,), jnp.float32)
    for j in range(STRIPE // 448):
        pltpu.sync_copy(zb, acc0.at[pl.ds(sid * STRIPE + j * 448, 448)])
        pltpu.sync_copy(zb, acc1.at[pl.ds(sid * STRIPE + j * 448, 448)])
    plsc.subcore_barrier()

    def chunk(i, carry):
        base = wid * t_edges + i * CH
        pltpu.sync_copy(src.at[pl.ds(base, CH)], idx_s.at[0])
        pltpu.sync_copy(dst.at[pl.ds(base, CH)], idx_d.at[0])
        for (asrc_h, adst_h, exp_h, acc_h) in ((asrc0, adst0, exp0, acc0),
                                               (asrc1, adst1, exp1, acc1)):
            pltpu.async_copy(asrc_h.at[idx_s.at[0]], a_v, sem).wait()
            pltpu.async_copy(adst_h.at[idx_d.at[0]], b_v, sem).wait()
            for k in range(CH // 16):
                sl = pl.ds(k * 16, 16)
                e_v[sl] = _leaky_exp(a_v[sl] + b_v[sl])
            pltpu.sync_copy(e_v, exp_h.at[pl.ds(base, CH)])
            pltpu.sync_copy(e_v, acc_h.at[idx_d.at[0]], add=True)
        return carry

    lax.fori_loop(0, t_edges // CH, chunk, 0)
    plsc.subcore_barrier()

    for acc_h, dh in ((acc0, dh0), (acc1, dh1)):
        for j in range(STRIPE // 448):
            off = sid * STRIPE + j * 448
            pltpu.sync_copy(acc_h.at[pl.ds(off, 448)], zb)
            pltpu.sync_copy(zb, dh.at[pl.ds(cid * NPAD + off, 448)])


def _sc_alpha_denom(asrc0, adst0, asrc1, adst1, src, dst):
    mesh = plsc.VectorSubcoreMesh(core_axis_name="c", subcore_axis_name="s")
    f32 = jnp.float32
    out = pl.kernel(
        _k1_body,
        mesh=mesh,
        out_type=[jax.ShapeDtypeStruct((EP,), f32),
                  jax.ShapeDtypeStruct((EP,), f32),
                  jax.ShapeDtypeStruct((2 * NPAD,), f32),
                  jax.ShapeDtypeStruct((2 * NPAD,), f32)],
        scratch_types=[pltpu.VMEM((1, CH), jnp.int32),
                       pltpu.VMEM((1, CH), jnp.int32),
                       pltpu.VMEM((CH,), f32),
                       pltpu.VMEM((CH,), f32),
                       pltpu.VMEM((CH,), f32),
                       pltpu.VMEM((448,), f32),
                       pltpu.VMEM_SHARED((NPAD,), f32),
                       pltpu.VMEM_SHARED((NPAD,), f32),
                       pltpu.SemaphoreType.DMA],
    )(asrc0, adst0, asrc1, adst1, src, dst)
    return out


def _make_k2_body(n_rounds):
    def _k2_body(*refs):
        hhp = refs[0:n_rounds]
        expp = refs[n_rounds:2 * n_rounds]
        denp = refs[2 * n_rounds:3 * n_rounds]
        src = refs[3 * n_rounds]
        dst = refs[3 * n_rounds + 1]
        outs = refs[3 * n_rounds + 2:4 * n_rounds + 2]
        idx_s, idx_d, idx_g, rows_v, d_v, e_v, zb, st, acc, sem = refs[4 * n_rounds + 2:]

        cid = lax.axis_index("c")
        sid = lax.axis_index("s")
        t_edges = EP // 16
        off_n = cid * NPAD
        off_e = cid * EP

        for k in range(64):
            for c2 in range(2):
                zb[k, pl.ds(c2 * 16, 16)] = jnp.zeros((16,), jnp.float32)

        for r in range(n_rounds):
            for j in range(STRIPE // 64):
                pltpu.sync_copy(zb, acc.at[pl.ds(sid * STRIPE + j * 64, 64), :])
            plsc.subcore_barrier()

            def chunk(i, carry, hh_s=hhp[r], exp_s=expp[r], den_s=denp[r],
                      ):
                base = sid * t_edges + i * CH
                pltpu.sync_copy(src.at[pl.ds(base, CH)], idx_s.at[0])
                pltpu.sync_copy(dst.at[pl.ds(base, CH)], idx_d.at[0])
                for k in range(CH // 16):
                    sl = pl.ds(k * 16, 16)
                    idx_s[0, sl] = idx_s[0, sl] + off_n
                    idx_g[0, sl] = idx_d[0, sl] + off_n
                pltpu.async_copy(hh_s.at[idx_s.at[0]], rows_v, sem).wait()
                pltpu.async_copy(den_s.at[idx_g.at[0]], d_v, sem).wait()
                pltpu.sync_copy(exp_s.at[pl.ds(off_e + base, CH)], e_v)
                for k in range(CH // 16):
                    sl = pl.ds(k * 16, 16)
                    wv = e_v[sl] / (d_v[sl] + 1e-16)
                    for j in range(16):
                        e = k * 16 + j
                        w = wv[j]
                        rows_v[e, pl.ds(0, 16)] = rows_v[e, pl.ds(0, 16)] * w
                        rows_v[e, pl.ds(16, 16)] = rows_v[e, pl.ds(16, 16)] * w
                pltpu.sync_copy(rows_v, acc.at[idx_d.at[0]], add=True)
                return carry

            lax.fori_loop(0, t_edges // CH, chunk, 0)
            plsc.subcore_barrier()

            for j in range(STRIPE // 64):
                off = sid * STRIPE + j * 64
                pltpu.sync_copy(acc.at[pl.ds(off, 64), :], st)
                pltpu.sync_copy(st, outs[r].at[pl.ds(off_n + off, 64), :])
            if r + 1 < n_rounds:
                plsc.subcore_barrier()

    return _k2_body


def _sc_aggregate(hh_pairs, exp_pairs, den_pairs, src, dst):
    n_rounds = len(hh_pairs)
    mesh = plsc.VectorSubcoreMesh(core_axis_name="c", subcore_axis_name="s")
    f32 = jnp.float32
    outs = pl.kernel(
        _make_k2_body(n_rounds),
        mesh=mesh,
        compiler_params=pltpu.CompilerParams(use_tc_tiling_on_sc=False),
        out_type=[jax.ShapeDtypeStruct((2 * NPAD, 32), f32)] * n_rounds,
        scratch_types=[pltpu.VMEM((1, CH), jnp.int32),
                       pltpu.VMEM((1, CH), jnp.int32),
                       pltpu.VMEM((1, CH), jnp.int32),
                       pltpu.VMEM((CH, 32), f32),
                       pltpu.VMEM((CH,), f32),
                       pltpu.VMEM((CH,), f32),
                       pltpu.VMEM((64, 32), f32),
                       pltpu.VMEM((64, 32), f32),
                       pltpu.VMEM_SHARED((NPAD, 32), f32),
                       pltpu.SemaphoreType.DMA],
    )(*hh_pairs, *exp_pairs, *den_pairs, src, dst)
    return outs if isinstance(outs, (list, tuple)) else [outs]


def _stage1_body(x_ref, w_in_ref, b_in_ref, g_ref, be_ref,
                 W0_ref, as0_ref, ad0_ref, Wr0_ref, br0_ref,
                 h_ref, hh0_ref, asrc_ref, adst_ref, res_ref):
    x = x_ref[...]
    w_in = w_in_ref[...]
    h = x * w_in + b_in_ref[...]
    m = jnp.mean(h, axis=-1, keepdims=True)
    v = jnp.mean((h - m) ** 2, axis=-1, keepdims=True)
    h = (h - m) * jax.lax.rsqrt(v + 1e-5) * g_ref[...] + be_ref[...]
    h = jnp.maximum(h, 0.0)
    h_ref[...] = h
    hh0 = jnp.dot(h, W0_ref[...], preferred_element_type=jnp.float32)
    hh0_ref[...] = hh0
    hh3 = hh0.reshape(h.shape[0], HEADS, 32)
    asrc_ref[...] = jnp.sum(hh3 * as0_ref[...][None], axis=-1)
    adst_ref[...] = jnp.sum(hh3 * ad0_ref[...][None], axis=-1)
    res_ref[...] = jnp.dot(h, Wr0_ref[...], preferred_element_type=jnp.float32) + br0_ref[...]


def _stage2_body(agg_ref, res_ref, bias0_ref,
                 W1_ref, as1_ref, ad1_ref, Wr1_ref, br1_ref,
                 hh1_ref, asrc_ref, adst_ref, res1_ref):
    x1 = jnp.maximum(agg_ref[...] + bias0_ref[...] + res_ref[...], 0.0)
    hh1 = jnp.dot(x1, W1_ref[...], preferred_element_type=jnp.float32)
    hh1_ref[...] = hh1
    hh3 = hh1.reshape(x1.shape[0], HEADS, 64)
    asrc_ref[...] = jnp.sum(hh3 * as1_ref[...][None], axis=-1)
    adst_ref[...] = jnp.sum(hh3 * ad1_ref[...][None], axis=-1)
    res1_ref[...] = jnp.dot(x1, Wr1_ref[...], preferred_element_type=jnp.float32) + br1_ref[...]


def _stage3_body(agg_ref, res_ref, bias1_ref, batch_ref,
                 Wc1_ref, bc1_ref, Wc2_ref, bc2_ref,
                 h_ref, clusters_ref):
    i = pl.program_id(0)
    h = jnp.maximum(agg_ref[...] + bias1_ref[...] + res_ref[...], 0.0)
    h_ref[...] = h
    a = jnp.dot(jnp.dot(h, Wc1_ref[...], preferred_element_type=jnp.float32) + bc1_ref[...],
                Wc2_ref[...], preferred_element_type=jnp.float32) + bc2_ref[...]
    a = a - jnp.max(a, axis=-1, keepdims=True)
    e = jnp.exp(a)
    assign = e / jnp.sum(e, axis=-1, keepdims=True)
    b = batch_ref[...].reshape(h.shape[0])
    onehot = (b[:, None] == jax.lax.broadcasted_iota(jnp.int32, (1, 8), 1)).astype(jnp.float32)
    w = onehot[:, :, None] * assign[:, None, :]
    w2 = w.reshape(h.shape[0], 64)
    part = jnp.dot(w2.T, h, preferred_element_type=jnp.float32)

    @pl.when(i == 0)
    def _():
        clusters_ref[...] = jnp.zeros_like(clusters_ref)

    clusters_ref[...] += part


def _dense_stage1(x, w_in, b_in, g_in, be_in, W0, as0, ad0, Wr0, br0):
    N = x.shape[0]
    grid = (N // N_BLOCK,)
    bs = lambda c: pl.BlockSpec((N_BLOCK, c), lambda i: (i, 0))
    ws = lambda shape: pl.BlockSpec(shape, lambda i: tuple(0 for _ in shape))
    return pl.pallas_call(
        _stage1_body,
        grid=grid,
        in_specs=[bs(1), ws((1, 64)), ws((64,)), ws((64,)), ws((64,)),
                  ws((64, 64)), ws((HEADS, 32)), ws((HEADS, 32)), ws((64, 64)), ws((64,))],
        out_specs=[bs(64), bs(64), bs(HEADS), bs(HEADS), bs(64)],
        out_shape=[jax.ShapeDtypeStruct((N, 64), jnp.float32),
                   jax.ShapeDtypeStruct((N, 64), jnp.float32),
                   jax.ShapeDtypeStruct((N, HEADS), jnp.float32),
                   jax.ShapeDtypeStruct((N, HEADS), jnp.float32),
                   jax.ShapeDtypeStruct((N, 64), jnp.float32)],
    )(x, w_in, b_in, g_in, be_in, W0, as0, ad0, Wr0, br0)


def _dense_stage2(agg, res, bias0, W1, as1, ad1, Wr1, br1):
    N = agg.shape[0]
    grid = (N // N_BLOCK,)
    bs = lambda c: pl.BlockSpec((N_BLOCK, c), lambda i: (i, 0))
    ws = lambda shape: pl.BlockSpec(shape, lambda i: tuple(0 for _ in shape))
    return pl.pallas_call(
        _stage2_body,
        grid=grid,
        in_specs=[bs(64), bs(64), ws((64,)),
                  ws((64, 128)), ws((HEADS, 64)), ws((HEADS, 64)), ws((64, 64)), ws((64,))],
        out_specs=[bs(128), bs(HEADS), bs(HEADS), bs(64)],
        out_shape=[jax.ShapeDtypeStruct((N, 128), jnp.float32),
                   jax.ShapeDtypeStruct((N, HEADS), jnp.float32),
                   jax.ShapeDtypeStruct((N, HEADS), jnp.float32),
                   jax.ShapeDtypeStruct((N, 64), jnp.float32)],
    )(agg, res, bias0, W1, as1, ad1, Wr1, br1)


def _dense_stage3(agg, res, bias1, batch, Wc1, bc1, Wc2, bc2):
    N = agg.shape[0]
    grid = (N // N_BLOCK,)
    bs = lambda c: pl.BlockSpec((N_BLOCK, c), lambda i: (i, 0))
    ws = lambda shape: pl.BlockSpec(shape, lambda i: tuple(0 for _ in shape))
    batch3 = batch.reshape(N // N_BLOCK, 1, N_BLOCK)
    h, clusters = pl.pallas_call(
        _stage3_body,
        grid=grid,
        in_specs=[bs(64), bs(64), ws((64,)),
                  pl.BlockSpec((1, 1, N_BLOCK), lambda i: (i, 0, 0)),
                  ws((64, 32)), ws((32,)), ws((32, 8)), ws((8,))],
        out_specs=[bs(64), ws((64, 64))],
        out_shape=[jax.ShapeDtypeStruct((N, 64), jnp.float32),
                   jax.ShapeDtypeStruct((64, 64), jnp.float32)],
    )(agg, res, bias1, batch3, Wc1, bc1, Wc2, bc2)
    return h, clusters


def _edge_softmax_agg(hh, asrc, adst, src, dst, N, out_ch):
    alpha = asrc[src] + adst[dst]
    alpha = jax.nn.leaky_relu(alpha, 0.2)
    amax = jax.ops.segment_max(alpha, dst, num_segments=N)
    amax = jnp.where(jnp.isfinite(amax), amax, 0.0)
    alpha = jnp.exp(alpha - amax[dst])
    denom = jax.ops.segment_sum(alpha, dst, num_segments=N)
    alpha = alpha / (denom[dst] + 1e-16)
    h3 = hh.reshape(N, HEADS, out_ch)
    msg = h3[src] * alpha[..., None]
    return jax.ops.segment_sum(msg, dst, num_segments=N)


def _final_head(clusters, Wd, bd, gd, bed, Wo, bo):
    c = clusters.reshape(8, 8, 64)
    emb = jnp.concatenate([c.max(axis=1), c.min(axis=1)], axis=1)
    e = jnp.dot(emb, Wd) + bd
    m = e.mean(-1, keepdims=True)
    v = ((e - m) ** 2).mean(-1, keepdims=True)
    e = (e - m) * jax.lax.rsqrt(v + 1e-5) * gd + bed
    e = jnp.where(e >= 0, e, 0.1 * e)
    return jnp.dot(e, Wo) + bo


def kernel(x, edge_index, batch, w_in, b_in, g_in, be_in, W0, as0, ad0, bias0,
           W1, as1, ad1, bias1, Wr0, br0, Wr1, br1, Wc1, bc1, Wc2, bc2,
           Wd, bd, gd, bed, Wo, bo):
    N = x.shape[0]
    sl = jnp.arange(N, dtype=edge_index.dtype)
    src = jnp.concatenate([edge_index[0], sl])
    dst = jnp.concatenate([edge_index[1], sl])
    src = jnp.pad(src, (0, EP - E_RAW), constant_values=DUMMY)
    dst = jnp.pad(dst, (0, EP - E_RAW), constant_values=DUMMY)
    pad_n = lambda a: jnp.pad(a, ((0, NPAD - N),) + ((0, 0),) * (a.ndim - 1))

    cat = jnp.concatenate
    h0, hh0, asrc0, adst0, res0 = _dense_stage1(
        x, w_in, b_in, g_in, be_in, W0, as0, ad0, Wr0, br0)
    asp, adp = pad_n(asrc0), pad_n(adst0)
    exp0, exp1, dh0, dh1 = _sc_alpha_denom(
        asp[:, 0], adp[:, 0], asp[:, 1], adp[:, 1], src, dst)
    den0 = dh0[:NPAD] + dh0[NPAD:]
    den1 = dh1[:NPAD] + dh1[NPAD:]
    (op0,) = _sc_aggregate(
        [cat([pad_n(hh0[:, :32]), pad_n(hh0[:, 32:])], axis=0)],
        [cat([exp0, exp1])], [cat([den0, den1])], src, dst)
    agg0 = cat([op0[:N], op0[NPAD:NPAD + N]], axis=1)

    hh1, asrc1, adst1, res1 = _dense_stage2(agg0, res0, bias0, W1, as1, ad1, Wr1, br1)
    asp, adp = pad_n(asrc1), pad_n(adst1)
    exp0, exp1, dh0, dh1 = _sc_alpha_denom(
        asp[:, 0], adp[:, 0], asp[:, 1], adp[:, 1], src, dst)
    den0 = dh0[:NPAD] + dh0[NPAD:]
    den1 = dh1[:NPAD] + dh1[NPAD:]
    o_r0, o_r1 = _sc_aggregate(
        [cat([pad_n(hh1[:, 0:32]), pad_n(hh1[:, 32:64])], axis=0),
         cat([pad_n(hh1[:, 64:96]), pad_n(hh1[:, 96:128])], axis=0)],
        [cat([exp0, exp0]), cat([exp1, exp1])],
        [cat([den0, den0]), cat([den1, den1])], src, dst)
    agg1 = 0.5 * (cat([o_r0[:N], o_r0[NPAD:NPAD + N]], axis=1) +
                  cat([o_r1[:N], o_r1[NPAD:NPAD + N]], axis=1))

    h2, clusters = _dense_stage3(agg1, res1, bias1, batch, Wc1, bc1, Wc2, bc2)
    return _final_head(clusters, Wd, bd, gd, bed, Wo, bo)


# speedup vs baseline: 40.1038x; 1.4188x over previous
import functools

import jax
import jax.numpy as jnp
from jax import lax
from jax.experimental import pallas as pl
from jax.experimental.pallas import tpu as pltpu
from jax.experimental.pallas import tpu_sc as plsc

HEADS = 2
N_BLOCK = 2000

N_NODES = 50000
NPAD = 50176
STRIPE = NPAD // 16
E_RAW = 800000 + N_NODES
EP = 851968
CH = 128
DUMMY = N_NODES


def _leaky_exp(t):
    return jnp.exp(jnp.where(t >= 0, t, 0.2 * t))


def _k1_body(asrc0, adst0, asrc1, adst1, src, dst,
             exp0, exp1, dh0, dh1,
             idx_s, idx_d, a_v, b_v, a2_v, b2_v, e_v, zb, acc0, acc1, sem):
    cid = lax.axis_index("c")
    sid = lax.axis_index("s")
    wid = sid * 2 + cid
    t_edges = EP // 32

    for k in range(448 // 16):
        zb[pl.ds(k * 16, 16)] = jnp.zeros((16,), jnp.float32)
    for j in range(STRIPE // 448):
        pltpu.sync_copy(zb, acc0.at[pl.ds(sid * STRIPE + j * 448, 448)])
        pltpu.sync_copy(zb, acc1.at[pl.ds(sid * STRIPE + j * 448, 448)])
    plsc.subcore_barrier()

    def chunk(i, carry):
        base = wid * t_edges + i * CH
        pltpu.sync_copy(src.at[pl.ds(base, CH)], idx_s.at[0])
        pltpu.sync_copy(dst.at[pl.ds(base, CH)], idx_d.at[0])
        c0 = pltpu.async_copy(asrc0.at[idx_s.at[0]], a_v, sem)
        c1 = pltpu.async_copy(adst0.at[idx_d.at[0]], b_v, sem)
        c2 = pltpu.async_copy(asrc1.at[idx_s.at[0]], a2_v, sem)
        c3 = pltpu.async_copy(adst1.at[idx_d.at[0]], b2_v, sem)
        c0.wait(); c1.wait(); c2.wait(); c3.wait()
        for (av, bv, exp_h, acc_h) in ((a_v, b_v, exp0, acc0),
                                       (a2_v, b2_v, exp1, acc1)):
            for k in range(CH // 16):
                sl = pl.ds(k * 16, 16)
                e_v[sl] = _leaky_exp(av[sl] + bv[sl])
            pltpu.sync_copy(e_v, exp_h.at[pl.ds(base, CH)])
            pltpu.sync_copy(e_v, acc_h.at[idx_d.at[0]], add=True)
        return carry

    lax.fori_loop(0, t_edges // CH, chunk, 0)
    plsc.subcore_barrier()

    for acc_h, dh in ((acc0, dh0), (acc1, dh1)):
        for j in range(STRIPE // 448):
            off = sid * STRIPE + j * 448
            pltpu.sync_copy(acc_h.at[pl.ds(off, 448)], zb)
            pltpu.sync_copy(zb, dh.at[pl.ds(cid * NPAD + off, 448)])


def _sc_alpha_denom(asrc0, adst0, asrc1, adst1, src, dst):
    mesh = plsc.VectorSubcoreMesh(core_axis_name="c", subcore_axis_name="s")
    f32 = jnp.float32
    out = pl.kernel(
        _k1_body,
        mesh=mesh,
        out_type=[jax.ShapeDtypeStruct((EP,), f32),
                  jax.ShapeDtypeStruct((EP,), f32),
                  jax.ShapeDtypeStruct((2 * NPAD,), f32),
                  jax.ShapeDtypeStruct((2 * NPAD,), f32)],
        scratch_types=[pltpu.VMEM((1, CH), jnp.int32),
                       pltpu.VMEM((1, CH), jnp.int32),
                       pltpu.VMEM((CH,), f32),
                       pltpu.VMEM((CH,), f32),
                       pltpu.VMEM((CH,), f32),
                       pltpu.VMEM((CH,), f32),
                       pltpu.VMEM((CH,), f32),
                       pltpu.VMEM((448,), f32),
                       pltpu.VMEM_SHARED((NPAD,), f32),
                       pltpu.VMEM_SHARED((NPAD,), f32),
                       pltpu.SemaphoreType.DMA],
    )(asrc0, adst0, asrc1, adst1, src, dst)
    return out


def _make_k2_body(n_rounds):
    def _k2_body(*refs):
        hhp = refs[0:n_rounds]
        expp = refs[n_rounds:2 * n_rounds]
        denp = refs[2 * n_rounds:3 * n_rounds]
        src = refs[3 * n_rounds]
        dst = refs[3 * n_rounds + 1]
        outs = refs[3 * n_rounds + 2:4 * n_rounds + 2]
        idx_s, idx_d, idx_g, rows_v, d_v, e_v, zb, st, acc, sem = refs[4 * n_rounds + 2:]

        cid = lax.axis_index("c")
        sid = lax.axis_index("s")
        t_edges = EP // 16
        off_n = cid * NPAD
        off_e = cid * EP

        for k in range(64):
            for c2 in range(2):
                zb[k, pl.ds(c2 * 16, 16)] = jnp.zeros((16,), jnp.float32)

        for r in range(n_rounds):
            for j in range(STRIPE // 64):
                pltpu.sync_copy(zb, acc.at[pl.ds(sid * STRIPE + j * 64, 64), :])
            plsc.subcore_barrier()

            def chunk(i, carry, hh_s=hhp[r], exp_s=expp[r], den_s=denp[r],
                      ):
                base = sid * t_edges + i * CH
                pltpu.sync_copy(src.at[pl.ds(base, CH)], idx_s.at[0])
                pltpu.sync_copy(dst.at[pl.ds(base, CH)], idx_d.at[0])
                for k in range(CH // 16):
                    sl = pl.ds(k * 16, 16)
                    idx_s[0, sl] = idx_s[0, sl] + off_n
                    idx_g[0, sl] = idx_d[0, sl] + off_n
                c0 = pltpu.async_copy(hh_s.at[idx_s.at[0]], rows_v, sem)
                c1 = pltpu.async_copy(den_s.at[idx_g.at[0]], d_v, sem)
                pltpu.sync_copy(exp_s.at[pl.ds(off_e + base, CH)], e_v)
                c0.wait(); c1.wait()
                for k in range(CH // 16):
                    sl = pl.ds(k * 16, 16)
                    wv = e_v[sl] / (d_v[sl] + 1e-16)
                    for j in range(16):
                        e = k * 16 + j
                        w = wv[j]
                        rows_v[e, pl.ds(0, 16)] = rows_v[e, pl.ds(0, 16)] * w
                        rows_v[e, pl.ds(16, 16)] = rows_v[e, pl.ds(16, 16)] * w
                pltpu.sync_copy(rows_v, acc.at[idx_d.at[0]], add=True)
                return carry

            lax.fori_loop(0, t_edges // CH, chunk, 0)
            plsc.subcore_barrier()

            for j in range(STRIPE // 64):
                off = sid * STRIPE + j * 64
                pltpu.sync_copy(acc.at[pl.ds(off, 64), :], st)
                pltpu.sync_copy(st, outs[r].at[pl.ds(off_n + off, 64), :])
            if r + 1 < n_rounds:
                plsc.subcore_barrier()

    return _k2_body


def _sc_aggregate(hh_pairs, exp_pairs, den_pairs, src, dst):
    n_rounds = len(hh_pairs)
    mesh = plsc.VectorSubcoreMesh(core_axis_name="c", subcore_axis_name="s")
    f32 = jnp.float32
    outs = pl.kernel(
        _make_k2_body(n_rounds),
        mesh=mesh,
        compiler_params=pltpu.CompilerParams(use_tc_tiling_on_sc=False),
        out_type=[jax.ShapeDtypeStruct((2 * NPAD, 32), f32)] * n_rounds,
        scratch_types=[pltpu.VMEM((1, CH), jnp.int32),
                       pltpu.VMEM((1, CH), jnp.int32),
                       pltpu.VMEM((1, CH), jnp.int32),
                       pltpu.VMEM((CH, 32), f32),
                       pltpu.VMEM((CH,), f32),
                       pltpu.VMEM((CH,), f32),
                       pltpu.VMEM((64, 32), f32),
                       pltpu.VMEM((64, 32), f32),
                       pltpu.VMEM_SHARED((NPAD, 32), f32),
                       pltpu.SemaphoreType.DMA],
    )(*hh_pairs, *exp_pairs, *den_pairs, src, dst)
    return outs if isinstance(outs, (list, tuple)) else [outs]


def _stage1_body(x_ref, w_in_ref, b_in_ref, g_ref, be_ref,
                 W0_ref, as0_ref, ad0_ref, Wr0_ref, br0_ref,
                 h_ref, hh0_ref, asrc_ref, adst_ref, res_ref):
    x = x_ref[...]
    w_in = w_in_ref[...]
    h = x * w_in + b_in_ref[...]
    m = jnp.mean(h, axis=-1, keepdims=True)
    v = jnp.mean((h - m) ** 2, axis=-1, keepdims=True)
    h = (h - m) * jax.lax.rsqrt(v + 1e-5) * g_ref[...] + be_ref[...]
    h = jnp.maximum(h, 0.0)
    h_ref[...] = h
    hh0 = jnp.dot(h, W0_ref[...], preferred_element_type=jnp.float32)
    hh0_ref[...] = hh0
    hh3 = hh0.reshape(h.shape[0], HEADS, 32)
    asrc_ref[...] = jnp.sum(hh3 * as0_ref[...][None], axis=-1)
    adst_ref[...] = jnp.sum(hh3 * ad0_ref[...][None], axis=-1)
    res_ref[...] = jnp.dot(h, Wr0_ref[...], preferred_element_type=jnp.float32) + br0_ref[...]


def _stage2_body(agg_ref, res_ref, bias0_ref,
                 W1_ref, as1_ref, ad1_ref, Wr1_ref, br1_ref,
                 hh1_ref, asrc_ref, adst_ref, res1_ref):
    x1 = jnp.maximum(agg_ref[...] + bias0_ref[...] + res_ref[...], 0.0)
    hh1 = jnp.dot(x1, W1_ref[...], preferred_element_type=jnp.float32)
    hh1_ref[...] = hh1
    hh3 = hh1.reshape(x1.shape[0], HEADS, 64)
    asrc_ref[...] = jnp.sum(hh3 * as1_ref[...][None], axis=-1)
    adst_ref[...] = jnp.sum(hh3 * ad1_ref[...][None], axis=-1)
    res1_ref[...] = jnp.dot(x1, Wr1_ref[...], preferred_element_type=jnp.float32) + br1_ref[...]


def _stage3_body(agg_ref, res_ref, bias1_ref, batch_ref,
                 Wc1_ref, bc1_ref, Wc2_ref, bc2_ref,
                 h_ref, clusters_ref):
    i = pl.program_id(0)
    h = jnp.maximum(agg_ref[...] + bias1_ref[...] + res_ref[...], 0.0)
    h_ref[...] = h
    a = jnp.dot(jnp.dot(h, Wc1_ref[...], preferred_element_type=jnp.float32) + bc1_ref[...],
                Wc2_ref[...], preferred_element_type=jnp.float32) + bc2_ref[...]
    a = a - jnp.max(a, axis=-1, keepdims=True)
    e = jnp.exp(a)
    assign = e / jnp.sum(e, axis=-1, keepdims=True)
    b = batch_ref[...].reshape(h.shape[0])
    onehot = (b[:, None] == jax.lax.broadcasted_iota(jnp.int32, (1, 8), 1)).astype(jnp.float32)
    w = onehot[:, :, None] * assign[:, None, :]
    w2 = w.reshape(h.shape[0], 64)
    part = jnp.dot(w2.T, h, preferred_element_type=jnp.float32)

    @pl.when(i == 0)
    def _():
        clusters_ref[...] = jnp.zeros_like(clusters_ref)

    clusters_ref[...] += part


def _dense_stage1(x, w_in, b_in, g_in, be_in, W0, as0, ad0, Wr0, br0):
    N = x.shape[0]
    grid = (N // N_BLOCK,)
    bs = lambda c: pl.BlockSpec((N_BLOCK, c), lambda i: (i, 0))
    ws = lambda shape: pl.BlockSpec(shape, lambda i: tuple(0 for _ in shape))
    return pl.pallas_call(
        _stage1_body,
        grid=grid,
        in_specs=[bs(1), ws((1, 64)), ws((64,)), ws((64,)), ws((64,)),
                  ws((64, 64)), ws((HEADS, 32)), ws((HEADS, 32)), ws((64, 64)), ws((64,))],
        out_specs=[bs(64), bs(64), bs(HEADS), bs(HEADS), bs(64)],
        out_shape=[jax.ShapeDtypeStruct((N, 64), jnp.float32),
                   jax.ShapeDtypeStruct((N, 64), jnp.float32),
                   jax.ShapeDtypeStruct((N, HEADS), jnp.float32),
                   jax.ShapeDtypeStruct((N, HEADS), jnp.float32),
                   jax.ShapeDtypeStruct((N, 64), jnp.float32)],
    )(x, w_in, b_in, g_in, be_in, W0, as0, ad0, Wr0, br0)


def _dense_stage2(agg, res, bias0, W1, as1, ad1, Wr1, br1):
    N = agg.shape[0]
    grid = (N // N_BLOCK,)
    bs = lambda c: pl.BlockSpec((N_BLOCK, c), lambda i: (i, 0))
    ws = lambda shape: pl.BlockSpec(shape, lambda i: tuple(0 for _ in shape))
    return pl.pallas_call(
        _stage2_body,
        grid=grid,
        in_specs=[bs(64), bs(64), ws((64,)),
                  ws((64, 128)), ws((HEADS, 64)), ws((HEADS, 64)), ws((64, 64)), ws((64,))],
        out_specs=[bs(128), bs(HEADS), bs(HEADS), bs(64)],
        out_shape=[jax.ShapeDtypeStruct((N, 128), jnp.float32),
                   jax.ShapeDtypeStruct((N, HEADS), jnp.float32),
                   jax.ShapeDtypeStruct((N, HEADS), jnp.float32),
                   jax.ShapeDtypeStruct((N, 64), jnp.float32)],
    )(agg, res, bias0, W1, as1, ad1, Wr1, br1)


def _dense_stage3(agg, res, bias1, batch, Wc1, bc1, Wc2, bc2):
    N = agg.shape[0]
    grid = (N // N_BLOCK,)
    bs = lambda c: pl.BlockSpec((N_BLOCK, c), lambda i: (i, 0))
    ws = lambda shape: pl.BlockSpec(shape, lambda i: tuple(0 for _ in shape))
    batch3 = batch.reshape(N // N_BLOCK, 1, N_BLOCK)
    h, clusters = pl.pallas_call(
        _stage3_body,
        grid=grid,
        in_specs=[bs(64), bs(64), ws((64,)),
                  pl.BlockSpec((1, 1, N_BLOCK), lambda i: (i, 0, 0)),
                  ws((64, 32)), ws((32,)), ws((32, 8)), ws((8,))],
        out_specs=[bs(64), ws((64, 64))],
        out_shape=[jax.ShapeDtypeStruct((N, 64), jnp.float32),
                   jax.ShapeDtypeStruct((64, 64), jnp.float32)],
    )(agg, res, bias1, batch3, Wc1, bc1, Wc2, bc2)
    return h, clusters


def _edge_softmax_agg(hh, asrc, adst, src, dst, N, out_ch):
    alpha = asrc[src] + adst[dst]
    alpha = jax.nn.leaky_relu(alpha, 0.2)
    amax = jax.ops.segment_max(alpha, dst, num_segments=N)
    amax = jnp.where(jnp.isfinite(amax), amax, 0.0)
    alpha = jnp.exp(alpha - amax[dst])
    denom = jax.ops.segment_sum(alpha, dst, num_segments=N)
    alpha = alpha / (denom[dst] + 1e-16)
    h3 = hh.reshape(N, HEADS, out_ch)
    msg = h3[src] * alpha[..., None]
    return jax.ops.segment_sum(msg, dst, num_segments=N)


def _final_head(clusters, Wd, bd, gd, bed, Wo, bo):
    c = clusters.reshape(8, 8, 64)
    emb = jnp.concatenate([c.max(axis=1), c.min(axis=1)], axis=1)
    e = jnp.dot(emb, Wd) + bd
    m = e.mean(-1, keepdims=True)
    v = ((e - m) ** 2).mean(-1, keepdims=True)
    e = (e - m) * jax.lax.rsqrt(v + 1e-5) * gd + bed
    e = jnp.where(e >= 0, e, 0.1 * e)
    return jnp.dot(e, Wo) + bo


def kernel(x, edge_index, batch, w_in, b_in, g_in, be_in, W0, as0, ad0, bias0,
           W1, as1, ad1, bias1, Wr0, br0, Wr1, br1, Wc1, bc1, Wc2, bc2,
           Wd, bd, gd, bed, Wo, bo):
    N = x.shape[0]
    sl = jnp.arange(N, dtype=edge_index.dtype)
    src = jnp.concatenate([edge_index[0], sl])
    dst = jnp.concatenate([edge_index[1], sl])
    src = jnp.pad(src, (0, EP - E_RAW), constant_values=DUMMY)
    dst = jnp.pad(dst, (0, EP - E_RAW), constant_values=DUMMY)
    pad_n = lambda a: jnp.pad(a, ((0, NPAD - N),) + ((0, 0),) * (a.ndim - 1))

    cat = jnp.concatenate
    h0, hh0, asrc0, adst0, res0 = _dense_stage1(
        x, w_in, b_in, g_in, be_in, W0, as0, ad0, Wr0, br0)
    asp, adp = pad_n(asrc0), pad_n(adst0)
    exp0, exp1, dh0, dh1 = _sc_alpha_denom(
        asp[:, 0], adp[:, 0], asp[:, 1], adp[:, 1], src, dst)
    den0 = dh0[:NPAD] + dh0[NPAD:]
    den1 = dh1[:NPAD] + dh1[NPAD:]
    (op0,) = _sc_aggregate(
        [cat([pad_n(hh0[:, :32]), pad_n(hh0[:, 32:])], axis=0)],
        [cat([exp0, exp1])], [cat([den0, den1])], src, dst)
    agg0 = cat([op0[:N], op0[NPAD:NPAD + N]], axis=1)

    hh1, asrc1, adst1, res1 = _dense_stage2(agg0, res0, bias0, W1, as1, ad1, Wr1, br1)
    asp, adp = pad_n(asrc1), pad_n(adst1)
    exp0, exp1, dh0, dh1 = _sc_alpha_denom(
        asp[:, 0], adp[:, 0], asp[:, 1], adp[:, 1], src, dst)
    den0 = dh0[:NPAD] + dh0[NPAD:]
    den1 = dh1[:NPAD] + dh1[NPAD:]
    o_r0, o_r1 = _sc_aggregate(
        [cat([pad_n(hh1[:, 0:32]), pad_n(hh1[:, 32:64])], axis=0),
         cat([pad_n(hh1[:, 64:96]), pad_n(hh1[:, 96:128])], axis=0)],
        [cat([exp0, exp0]), cat([exp1, exp1])],
        [cat([den0, den0]), cat([den1, den1])], src, dst)
    agg1 = 0.5 * (cat([o_r0[:N], o_r0[NPAD:NPAD + N]], axis=1) +
                  cat([o_r1[:N], o_r1[NPAD:NPAD + N]], axis=1))

    h2, clusters = _dense_stage3(agg1, res1, bias1, batch, Wc1, bc1, Wc2, bc2)
    return _final_head(clusters, Wd, bd, gd, bed, Wo, bo)


# speedup vs baseline: 44.2958x; 1.1045x over previous
import functools

import jax
import jax.numpy as jnp
from jax import lax
from jax.experimental import pallas as pl
from jax.experimental.pallas import tpu as pltpu
from jax.experimental.pallas import tpu_sc as plsc

HEADS = 2
N_BLOCK = 2000

N_NODES = 50000
NPAD = 50176
STRIPE = NPAD // 16
E_RAW = 800000 + N_NODES
EP = 851968
CH = 128
DUMMY = N_NODES


def _leaky_exp(t):
    return jnp.exp(jnp.where(t >= 0, t, 0.2 * t))


def _k1_body(asrc0, adst0, asrc1, adst1, src, dst,
             exp0, exp1, dh0, dh1,
             idx_s, idx_d, a_v, b_v, a2_v, b2_v, e_v, zb, acc0, acc1, sem):
    cid = lax.axis_index("c")
    sid = lax.axis_index("s")
    wid = sid * 2 + cid
    t_edges = EP // 32

    for k in range(448 // 16):
        zb[pl.ds(k * 16, 16)] = jnp.zeros((16,), jnp.float32)
    for j in range(STRIPE // 448):
        pltpu.sync_copy(zb, acc0.at[pl.ds(sid * STRIPE + j * 448, 448)])
        pltpu.sync_copy(zb, acc1.at[pl.ds(sid * STRIPE + j * 448, 448)])
    plsc.subcore_barrier()

    def chunk(i, carry):
        base = wid * t_edges + i * CH
        pltpu.sync_copy(src.at[pl.ds(base, CH)], idx_s.at[0])
        pltpu.sync_copy(dst.at[pl.ds(base, CH)], idx_d.at[0])
        c0 = pltpu.async_copy(asrc0.at[idx_s.at[0]], a_v, sem)
        c1 = pltpu.async_copy(adst0.at[idx_d.at[0]], b_v, sem)
        c2 = pltpu.async_copy(asrc1.at[idx_s.at[0]], a2_v, sem)
        c3 = pltpu.async_copy(adst1.at[idx_d.at[0]], b2_v, sem)
        c0.wait(); c1.wait(); c2.wait(); c3.wait()
        for (av, bv, exp_h, acc_h) in ((a_v, b_v, exp0, acc0),
                                       (a2_v, b2_v, exp1, acc1)):
            for k in range(CH // 16):
                sl = pl.ds(k * 16, 16)
                e_v[sl] = _leaky_exp(av[sl] + bv[sl])
            pltpu.sync_copy(e_v, exp_h.at[pl.ds(base, CH)])
            pltpu.sync_copy(e_v, acc_h.at[idx_d.at[0]], add=True)
        return carry

    lax.fori_loop(0, t_edges // CH, chunk, 0)
    plsc.subcore_barrier()

    for acc_h, dh in ((acc0, dh0), (acc1, dh1)):
        for j in range(STRIPE // 448):
            off = sid * STRIPE + j * 448
            pltpu.sync_copy(acc_h.at[pl.ds(off, 448)], zb)
            pltpu.sync_copy(zb, dh.at[pl.ds(cid * NPAD + off, 448)])


def _sc_alpha_denom(asrc0, adst0, asrc1, adst1, src, dst):
    mesh = plsc.VectorSubcoreMesh(core_axis_name="c", subcore_axis_name="s")
    f32 = jnp.float32
    out = pl.kernel(
        _k1_body,
        mesh=mesh,
        out_type=[jax.ShapeDtypeStruct((EP,), f32),
                  jax.ShapeDtypeStruct((EP,), f32),
                  jax.ShapeDtypeStruct((2 * NPAD,), f32),
                  jax.ShapeDtypeStruct((2 * NPAD,), f32)],
        scratch_types=[pltpu.VMEM((1, CH), jnp.int32),
                       pltpu.VMEM((1, CH), jnp.int32),
                       pltpu.VMEM((CH,), f32),
                       pltpu.VMEM((CH,), f32),
                       pltpu.VMEM((CH,), f32),
                       pltpu.VMEM((CH,), f32),
                       pltpu.VMEM((CH,), f32),
                       pltpu.VMEM((448,), f32),
                       pltpu.VMEM_SHARED((NPAD,), f32),
                       pltpu.VMEM_SHARED((NPAD,), f32),
                       pltpu.SemaphoreType.DMA],
    )(asrc0, adst0, asrc1, adst1, src, dst)
    return out


def _make_k2_body(n_rounds):
    def _k2_body(*refs):
        hhp = refs[0:n_rounds]
        expp = refs[n_rounds:2 * n_rounds]
        denp = refs[2 * n_rounds:3 * n_rounds]
        src = refs[3 * n_rounds]
        dst = refs[3 * n_rounds + 1]
        outs = refs[3 * n_rounds + 2:4 * n_rounds + 2]
        idx_s, idx_d, idx_g, rows_v, d_v, e_v, zb, st, acc, sem = refs[4 * n_rounds + 2:]

        cid = lax.axis_index("c")
        sid = lax.axis_index("s")
        t_edges = EP // 16
        off_n = cid * NPAD
        off_e = cid * EP

        for k in range(64):
            for c2 in range(2):
                zb[k, pl.ds(c2 * 16, 16)] = jnp.zeros((16,), jnp.float32)

        for r in range(n_rounds):
            for j in range(STRIPE // 64):
                pltpu.sync_copy(zb, acc.at[pl.ds(sid * STRIPE + j * 64, 64), :])
            plsc.subcore_barrier()

            def chunk(i, carry, hh_s=hhp[r], exp_s=expp[r], den_s=denp[r],
                      ):
                base = sid * t_edges + i * CH
                ci = pltpu.async_copy(src.at[pl.ds(base, CH)], idx_s.at[0], sem)
                cj = pltpu.async_copy(dst.at[pl.ds(base, CH)], idx_d.at[0], sem)
                ci.wait(); cj.wait()
                for k in range(CH // 16):
                    sl = pl.ds(k * 16, 16)
                    idx_s[0, sl] = idx_s[0, sl] + off_n
                    idx_g[0, sl] = idx_d[0, sl] + off_n
                c0 = pltpu.async_copy(hh_s.at[idx_s.at[0]], rows_v, sem)
                c1 = pltpu.async_copy(den_s.at[idx_g.at[0]], d_v, sem)
                pltpu.sync_copy(exp_s.at[pl.ds(off_e + base, CH)], e_v)
                c0.wait(); c1.wait()
                for k in range(CH // 16):
                    sl = pl.ds(k * 16, 16)
                    wv = e_v[sl] / (d_v[sl] + 1e-16)
                    for j in range(16):
                        e = k * 16 + j
                        w = wv[j]
                        rows_v[e, pl.ds(0, 16)] = rows_v[e, pl.ds(0, 16)] * w
                        rows_v[e, pl.ds(16, 16)] = rows_v[e, pl.ds(16, 16)] * w
                pltpu.sync_copy(rows_v, acc.at[idx_d.at[0]], add=True)
                return carry

            lax.fori_loop(0, t_edges // CH, chunk, 0)
            plsc.subcore_barrier()

            for j in range(STRIPE // 64):
                off = sid * STRIPE + j * 64
                pltpu.sync_copy(acc.at[pl.ds(off, 64), :], st)
                pltpu.sync_copy(st, outs[r].at[pl.ds(off_n + off, 64), :])
            if r + 1 < n_rounds:
                plsc.subcore_barrier()

    return _k2_body


def _sc_aggregate(hh_pairs, exp_pairs, den_pairs, src, dst):
    n_rounds = len(hh_pairs)
    mesh = plsc.VectorSubcoreMesh(core_axis_name="c", subcore_axis_name="s")
    f32 = jnp.float32
    outs = pl.kernel(
        _make_k2_body(n_rounds),
        mesh=mesh,
        compiler_params=pltpu.CompilerParams(use_tc_tiling_on_sc=False),
        out_type=[jax.ShapeDtypeStruct((2 * NPAD, 32), f32)] * n_rounds,
        scratch_types=[pltpu.VMEM((1, CH), jnp.int32),
                       pltpu.VMEM((1, CH), jnp.int32),
                       pltpu.VMEM((1, CH), jnp.int32),
                       pltpu.VMEM((CH, 32), f32),
                       pltpu.VMEM((CH,), f32),
                       pltpu.VMEM((CH,), f32),
                       pltpu.VMEM((64, 32), f32),
                       pltpu.VMEM((64, 32), f32),
                       pltpu.VMEM_SHARED((NPAD, 32), f32),
                       pltpu.SemaphoreType.DMA],
    )(*hh_pairs, *exp_pairs, *den_pairs, src, dst)
    return outs if isinstance(outs, (list, tuple)) else [outs]


def _stage1_body(x_ref, w_in_ref, b_in_ref, g_ref, be_ref,
                 W0_ref, as0_ref, ad0_ref, Wr0_ref, br0_ref,
                 h_ref, hh0_ref, asrc_ref, adst_ref, res_ref):
    x = x_ref[...]
    w_in = w_in_ref[...]
    h = x * w_in + b_in_ref[...]
    m = jnp.mean(h, axis=-1, keepdims=True)
    v = jnp.mean((h - m) ** 2, axis=-1, keepdims=True)
    h = (h - m) * jax.lax.rsqrt(v + 1e-5) * g_ref[...] + be_ref[...]
    h = jnp.maximum(h, 0.0)
    h_ref[...] = h
    hh0 = jnp.dot(h, W0_ref[...], preferred_element_type=jnp.float32)
    hh0_ref[...] = hh0
    hh3 = hh0.reshape(h.shape[0], HEADS, 32)
    asrc_ref[...] = jnp.sum(hh3 * as0_ref[...][None], axis=-1)
    adst_ref[...] = jnp.sum(hh3 * ad0_ref[...][None], axis=-1)
    res_ref[...] = jnp.dot(h, Wr0_ref[...], preferred_element_type=jnp.float32) + br0_ref[...]


def _stage2_body(agg_ref, res_ref, bias0_ref,
                 W1_ref, as1_ref, ad1_ref, Wr1_ref, br1_ref,
                 hh1_ref, asrc_ref, adst_ref, res1_ref):
    x1 = jnp.maximum(agg_ref[...] + bias0_ref[...] + res_ref[...], 0.0)
    hh1 = jnp.dot(x1, W1_ref[...], preferred_element_type=jnp.float32)
    hh1_ref[...] = hh1
    hh3 = hh1.reshape(x1.shape[0], HEADS, 64)
    asrc_ref[...] = jnp.sum(hh3 * as1_ref[...][None], axis=-1)
    adst_ref[...] = jnp.sum(hh3 * ad1_ref[...][None], axis=-1)
    res1_ref[...] = jnp.dot(x1, Wr1_ref[...], preferred_element_type=jnp.float32) + br1_ref[...]


def _stage3_body(agg_ref, res_ref, bias1_ref, batch_ref,
                 Wc1_ref, bc1_ref, Wc2_ref, bc2_ref,
                 h_ref, clusters_ref):
    i = pl.program_id(0)
    h = jnp.maximum(agg_ref[...] + bias1_ref[...] + res_ref[...], 0.0)
    h_ref[...] = h
    a = jnp.dot(jnp.dot(h, Wc1_ref[...], preferred_element_type=jnp.float32) + bc1_ref[...],
                Wc2_ref[...], preferred_element_type=jnp.float32) + bc2_ref[...]
    a = a - jnp.max(a, axis=-1, keepdims=True)
    e = jnp.exp(a)
    assign = e / jnp.sum(e, axis=-1, keepdims=True)
    b = batch_ref[...].reshape(h.shape[0])
    onehot = (b[:, None] == jax.lax.broadcasted_iota(jnp.int32, (1, 8), 1)).astype(jnp.float32)
    w = onehot[:, :, None] * assign[:, None, :]
    w2 = w.reshape(h.shape[0], 64)
    part = jnp.dot(w2.T, h, preferred_element_type=jnp.float32)

    @pl.when(i == 0)
    def _():
        clusters_ref[...] = jnp.zeros_like(clusters_ref)

    clusters_ref[...] += part


def _dense_stage1(x, w_in, b_in, g_in, be_in, W0, as0, ad0, Wr0, br0):
    N = x.shape[0]
    grid = (N // N_BLOCK,)
    bs = lambda c: pl.BlockSpec((N_BLOCK, c), lambda i: (i, 0))
    ws = lambda shape: pl.BlockSpec(shape, lambda i: tuple(0 for _ in shape))
    return pl.pallas_call(
        _stage1_body,
        grid=grid,
        in_specs=[bs(1), ws((1, 64)), ws((64,)), ws((64,)), ws((64,)),
                  ws((64, 64)), ws((HEADS, 32)), ws((HEADS, 32)), ws((64, 64)), ws((64,))],
        out_specs=[bs(64), bs(64), bs(HEADS), bs(HEADS), bs(64)],
        out_shape=[jax.ShapeDtypeStruct((N, 64), jnp.float32),
                   jax.ShapeDtypeStruct((N, 64), jnp.float32),
                   jax.ShapeDtypeStruct((N, HEADS), jnp.float32),
                   jax.ShapeDtypeStruct((N, HEADS), jnp.float32),
                   jax.ShapeDtypeStruct((N, 64), jnp.float32)],
    )(x, w_in, b_in, g_in, be_in, W0, as0, ad0, Wr0, br0)


def _dense_stage2(agg, res, bias0, W1, as1, ad1, Wr1, br1):
    N = agg.shape[0]
    grid = (N // N_BLOCK,)
    bs = lambda c: pl.BlockSpec((N_BLOCK, c), lambda i: (i, 0))
    ws = lambda shape: pl.BlockSpec(shape, lambda i: tuple(0 for _ in shape))
    return pl.pallas_call(
        _stage2_body,
        grid=grid,
        in_specs=[bs(64), bs(64), ws((64,)),
                  ws((64, 128)), ws((HEADS, 64)), ws((HEADS, 64)), ws((64, 64)), ws((64,))],
        out_specs=[bs(128), bs(HEADS), bs(HEADS), bs(64)],
        out_shape=[jax.ShapeDtypeStruct((N, 128), jnp.float32),
                   jax.ShapeDtypeStruct((N, HEADS), jnp.float32),
                   jax.ShapeDtypeStruct((N, HEADS), jnp.float32),
                   jax.ShapeDtypeStruct((N, 64), jnp.float32)],
    )(agg, res, bias0, W1, as1, ad1, Wr1, br1)


def _dense_stage3(agg, res, bias1, batch, Wc1, bc1, Wc2, bc2):
    N = agg.shape[0]
    grid = (N // N_BLOCK,)
    bs = lambda c: pl.BlockSpec((N_BLOCK, c), lambda i: (i, 0))
    ws = lambda shape: pl.BlockSpec(shape, lambda i: tuple(0 for _ in shape))
    batch3 = batch.reshape(N // N_BLOCK, 1, N_BLOCK)
    h, clusters = pl.pallas_call(
        _stage3_body,
        grid=grid,
        in_specs=[bs(64), bs(64), ws((64,)),
                  pl.BlockSpec((1, 1, N_BLOCK), lambda i: (i, 0, 0)),
                  ws((64, 32)), ws((32,)), ws((32, 8)), ws((8,))],
        out_specs=[bs(64), ws((64, 64))],
        out_shape=[jax.ShapeDtypeStruct((N, 64), jnp.float32),
                   jax.ShapeDtypeStruct((64, 64), jnp.float32)],
    )(agg, res, bias1, batch3, Wc1, bc1, Wc2, bc2)
    return h, clusters


def _edge_softmax_agg(hh, asrc, adst, src, dst, N, out_ch):
    alpha = asrc[src] + adst[dst]
    alpha = jax.nn.leaky_relu(alpha, 0.2)
    amax = jax.ops.segment_max(alpha, dst, num_segments=N)
    amax = jnp.where(jnp.isfinite(amax), amax, 0.0)
    alpha = jnp.exp(alpha - amax[dst])
    denom = jax.ops.segment_sum(alpha, dst, num_segments=N)
    alpha = alpha / (denom[dst] + 1e-16)
    h3 = hh.reshape(N, HEADS, out_ch)
    msg = h3[src] * alpha[..., None]
    return jax.ops.segment_sum(msg, dst, num_segments=N)


def _final_head(clusters, Wd, bd, gd, bed, Wo, bo):
    c = clusters.reshape(8, 8, 64)
    emb = jnp.concatenate([c.max(axis=1), c.min(axis=1)], axis=1)
    e = jnp.dot(emb, Wd) + bd
    m = e.mean(-1, keepdims=True)
    v = ((e - m) ** 2).mean(-1, keepdims=True)
    e = (e - m) * jax.lax.rsqrt(v + 1e-5) * gd + bed
    e = jnp.where(e >= 0, e, 0.1 * e)
    return jnp.dot(e, Wo) + bo


def kernel(x, edge_index, batch, w_in, b_in, g_in, be_in, W0, as0, ad0, bias0,
           W1, as1, ad1, bias1, Wr0, br0, Wr1, br1, Wc1, bc1, Wc2, bc2,
           Wd, bd, gd, bed, Wo, bo):
    N = x.shape[0]
    sl = jnp.arange(N, dtype=edge_index.dtype)
    src = jnp.concatenate([edge_index[0], sl])
    dst = jnp.concatenate([edge_index[1], sl])
    src = jnp.pad(src, (0, EP - E_RAW), constant_values=DUMMY)
    dst = jnp.pad(dst, (0, EP - E_RAW), constant_values=DUMMY)
    pad_n = lambda a: jnp.pad(a, ((0, NPAD - N),) + ((0, 0),) * (a.ndim - 1))

    cat = jnp.concatenate
    h0, hh0, asrc0, adst0, res0 = _dense_stage1(
        x, w_in, b_in, g_in, be_in, W0, as0, ad0, Wr0, br0)
    asp, adp = pad_n(asrc0), pad_n(adst0)
    exp0, exp1, dh0, dh1 = _sc_alpha_denom(
        asp[:, 0], adp[:, 0], asp[:, 1], adp[:, 1], src, dst)
    den0 = dh0[:NPAD] + dh0[NPAD:]
    den1 = dh1[:NPAD] + dh1[NPAD:]
    (op0,) = _sc_aggregate(
        [cat([pad_n(hh0[:, :32]), pad_n(hh0[:, 32:])], axis=0)],
        [cat([exp0, exp1])], [cat([den0, den1])], src, dst)
    agg0 = cat([op0[:N], op0[NPAD:NPAD + N]], axis=1)

    hh1, asrc1, adst1, res1 = _dense_stage2(agg0, res0, bias0, W1, as1, ad1, Wr1, br1)
    asp, adp = pad_n(asrc1), pad_n(adst1)
    exp0, exp1, dh0, dh1 = _sc_alpha_denom(
        asp[:, 0], adp[:, 0], asp[:, 1], adp[:, 1], src, dst)
    den0 = dh0[:NPAD] + dh0[NPAD:]
    den1 = dh1[:NPAD] + dh1[NPAD:]
    o_r0, o_r1 = _sc_aggregate(
        [cat([pad_n(hh1[:, 0:32]), pad_n(hh1[:, 32:64])], axis=0),
         cat([pad_n(hh1[:, 64:96]), pad_n(hh1[:, 96:128])], axis=0)],
        [cat([exp0, exp0]), cat([exp1, exp1])],
        [cat([den0, den0]), cat([den1, den1])], src, dst)
    agg1 = 0.5 * (cat([o_r0[:N], o_r0[NPAD:NPAD + N]], axis=1) +
                  cat([o_r1[:N], o_r1[NPAD:NPAD + N]], axis=1))

    h2, clusters = _dense_stage3(agg1, res1, bias1, batch, Wc1, bc1, Wc2, bc2)
    return _final_head(clusters, Wd, bd, gd, bed, Wo, bo)
